# baseline (speedup 1.0000x reference)
import functools
import math

import jax
import jax.numpy as jnp
from jax import lax
from jax.experimental import pallas as pl
from jax.experimental.pallas import tpu as pltpu
from jax.experimental.pallas import tpu_sc as plsc

EMB = 32
SCALE = math.sqrt(EMB)

NC, NS = 2, 16
NW = NC * NS
CH = 1024


def _scale_body(x_ref, o_ref):
    o_ref[...] = x_ref[...] * SCALE


def _scaled_table(table):
    v, e = table.shape
    blk = 8000
    return pl.pallas_call(
        _scale_body,
        grid=(v // blk,),
        in_specs=[pl.BlockSpec((blk, e), lambda i: (i, 0))],
        out_specs=pl.BlockSpec((blk, e), lambda i: (i, 0)),
        out_shape=jax.ShapeDtypeStruct((v, e), jnp.float32),
    )(table)


def _make_gather(B):
    b_per_w = B // NW
    n_chunks = b_per_w // CH
    mesh = plsc.VectorSubcoreMesh(core_axis_name="c", subcore_axis_name="s")

    @functools.partial(
        pl.kernel,
        mesh=mesh,
        out_type=jax.ShapeDtypeStruct((B, EMB), jnp.float32),
        scratch_types=[
            pltpu.VMEM((CH,), jnp.int32),
            pltpu.VMEM((CH, EMB), jnp.float32),
            pltpu.SemaphoreType.DMA,
        ],
        compiler_params=pltpu.CompilerParams(use_tc_tiling_on_sc=False),
    )
    def gather_kernel(idx_hbm, tab_hbm, out_hbm, idx_v, rows_v, sem):
        wid = lax.axis_index("s") * NC + lax.axis_index("c")
        base = wid * b_per_w

        def body(i, carry):
            off = base + i * CH
            pltpu.sync_copy(idx_hbm.at[pl.ds(off, CH)], idx_v)
            pltpu.async_copy(tab_hbm.at[idx_v], rows_v, sem).wait()
            pltpu.sync_copy(rows_v, out_hbm.at[pl.ds(off, CH)])
            return carry

        lax.fori_loop(0, n_chunks, body, 0)

    return gather_kernel


def kernel(tokens, table):
    B0, T = tokens.shape
    B = B0 * T
    scaled = _scaled_table(table)
    idx = tokens.reshape(B).astype(jnp.int32)
    out = _make_gather(B)(idx, scaled)
    return out.reshape(B0, T, EMB)


# speedup vs baseline: 1.2195x; 1.2195x over previous
import functools
import math

import jax
import jax.numpy as jnp
from jax import lax
from jax.experimental import pallas as pl
from jax.experimental.pallas import tpu as pltpu
from jax.experimental.pallas import tpu_sc as plsc

EMB = 32
SCALE = math.sqrt(EMB)

NC, NS = 2, 16
NW = NC * NS
CH = 1024


def _scale_rows(rows):
    def body(j, carry):
        rr = j * 8
        for t in range(8):
            for k in (0, 16):
                sl = (rr + t, pl.ds(k, 16))
                rows[sl] = rows[sl] * SCALE
        return carry

    lax.fori_loop(0, CH // 8, body, 0)


def _make_gather(B):
    b_per_w = B // NW
    n_pairs = b_per_w // (2 * CH)
    mesh = plsc.VectorSubcoreMesh(core_axis_name="c", subcore_axis_name="s")

    @functools.partial(
        pl.kernel,
        mesh=mesh,
        out_type=jax.ShapeDtypeStruct((B, EMB), jnp.float32),
        scratch_types=[
            pltpu.VMEM((CH,), jnp.int32),
            pltpu.VMEM((CH,), jnp.int32),
            pltpu.VMEM((CH, EMB), jnp.float32),
            pltpu.VMEM((CH, EMB), jnp.float32),
            pltpu.SemaphoreType.DMA,
            pltpu.SemaphoreType.DMA,
            pltpu.SemaphoreType.DMA,
            pltpu.SemaphoreType.DMA,
        ],
        compiler_params=pltpu.CompilerParams(use_tc_tiling_on_sc=False),
    )
    def gather_kernel(idx_hbm, tab_hbm, out_hbm,
                      idx0, idx1, rows0, rows1,
                      sem_g0, sem_g1, sem_o0, sem_o1):
        wid = lax.axis_index("s") * NC + lax.axis_index("c")
        base = wid * b_per_w

        def out_wait(rows, sem):
            pltpu.make_async_copy(rows, out_hbm.at[pl.ds(0, CH)], sem).wait()

        def body(g, carry):
            c0 = base + (2 * g) * CH
            c1 = c0 + CH
            pltpu.sync_copy(idx_hbm.at[pl.ds(c0, CH)], idx0)

            @pl.when(g > 0)
            def _():
                out_wait(rows0, sem_o0)

            gcopy0 = pltpu.make_async_copy(tab_hbm.at[idx0], rows0, sem_g0)
            gcopy0.start()

            @pl.when(g > 0)
            def _():
                _scale_rows(rows1)
                pltpu.async_copy(rows1, out_hbm.at[pl.ds(c0 - CH, CH)], sem_o1)

            gcopy0.wait()
            pltpu.sync_copy(idx_hbm.at[pl.ds(c1, CH)], idx1)

            @pl.when(g > 0)
            def _():
                out_wait(rows1, sem_o1)

            gcopy1 = pltpu.make_async_copy(tab_hbm.at[idx1], rows1, sem_g1)
            gcopy1.start()

            _scale_rows(rows0)
            pltpu.async_copy(rows0, out_hbm.at[pl.ds(c0, CH)], sem_o0)

            gcopy1.wait()
            return carry

        lax.fori_loop(0, n_pairs, body, 0)

        last = base + (2 * n_pairs - 1) * CH
        _scale_rows(rows1)
        pltpu.async_copy(rows1, out_hbm.at[pl.ds(last, CH)], sem_o1)
        out_wait(rows0, sem_o0)
        out_wait(rows1, sem_o1)

    return gather_kernel


def kernel(tokens, table):
    B0, T = tokens.shape
    B = B0 * T
    idx = tokens.reshape(B).astype(jnp.int32)
    out = _make_gather(B)(idx, table)
    return out.reshape(B0, T, EMB)


# speedup vs baseline: 1.2325x; 1.0106x over previous
import functools
import math

import jax
import jax.numpy as jnp
from jax import lax
from jax.experimental import pallas as pl
from jax.experimental.pallas import tpu as pltpu
from jax.experimental.pallas import tpu_sc as plsc

EMB = 32
SCALE = math.sqrt(EMB)

NC, NS = 2, 16
NW = NC * NS
K = 8


def _make_kernel(B0, T):
    rows_per_w = B0 // NW
    n_pairs = rows_per_w // (2 * K)
    pieces = [(0, 104), (104, T - 104)]
    mesh = plsc.VectorSubcoreMesh(core_axis_name="c", subcore_axis_name="s")

    @functools.partial(
        pl.kernel,
        mesh=mesh,
        out_type=jax.ShapeDtypeStruct((B0, T, EMB), jnp.float32),
        scratch_types=[
            pltpu.VMEM((K, T), jnp.int32),
            pltpu.VMEM((K, T), jnp.int32),
            pltpu.VMEM((K, T, EMB), jnp.float32),
            pltpu.VMEM((K, T, EMB), jnp.float32),
            pltpu.SemaphoreType.DMA,
            pltpu.SemaphoreType.DMA,
            pltpu.SemaphoreType.DMA,
            pltpu.SemaphoreType.DMA,
        ],
        compiler_params=pltpu.CompilerParams(use_tc_tiling_on_sc=False),
    )
    def body_kernel(tok_hbm, tab_hbm, out_hbm,
                    idx0, idx1, rows0, rows1,
                    sem_g0, sem_g1, sem_o0, sem_o1):
        wid = lax.axis_index("s") * NC + lax.axis_index("c")
        base = wid * rows_per_w

        def scale_rows(rows):
            for j in range(K):
                def sbody(i, carry, j=j):
                    r = i * 8
                    for t in range(8):
                        for h in (0, 16):
                            sl = (j, r + t, pl.ds(h, 16))
                            rows[sl] = rows[sl] * SCALE
                    return carry
                lax.fori_loop(0, T // 8, sbody, 0)

        def start_gathers(idx, rows, sem):
            copies = []
            for j in range(K):
                for off, ln in pieces:
                    c = pltpu.make_async_copy(
                        tab_hbm.at[idx.at[j, pl.ds(off, ln)]],
                        rows.at[j, pl.ds(off, ln)],
                        sem,
                    )
                    c.start()
                    copies.append(c)
            return copies

        def out_wait(rows, sem):
            pltpu.make_async_copy(rows, out_hbm.at[pl.ds(0, K)], sem).wait()

        def body(g, carry):
            r0 = base + (2 * g) * K
            r1 = r0 + K
            pltpu.sync_copy(tok_hbm.at[pl.ds(r0, K)], idx0)

            @pl.when(g > 0)
            def _():
                out_wait(rows0, sem_o0)

            g0 = start_gathers(idx0, rows0, sem_g0)

            @pl.when(g > 0)
            def _():
                scale_rows(rows1)
                pltpu.async_copy(rows1, out_hbm.at[pl.ds(r0 - K, K)], sem_o1)

            for c in g0:
                c.wait()
            pltpu.sync_copy(tok_hbm.at[pl.ds(r1, K)], idx1)

            @pl.when(g > 0)
            def _():
                out_wait(rows1, sem_o1)

            g1 = start_gathers(idx1, rows1, sem_g1)

            scale_rows(rows0)
            pltpu.async_copy(rows0, out_hbm.at[pl.ds(r0, K)], sem_o0)

            for c in g1:
                c.wait()
            return carry

        lax.fori_loop(0, n_pairs, body, 0)

        last = base + (2 * n_pairs - 1) * K
        scale_rows(rows1)
        pltpu.async_copy(rows1, out_hbm.at[pl.ds(last, K)], sem_o1)
        out_wait(rows0, sem_o0)
        out_wait(rows1, sem_o1)

    return body_kernel


def kernel(tokens, table):
    B0, T = tokens.shape
    return _make_kernel(B0, T)(tokens.astype(jnp.int32), table)
